# baseline (speedup 1.0000x reference)
import jax
import jax.numpy as jnp
from jax.experimental import pallas as pl
from jax.experimental.pallas import tpu as pltpu

D_MODEL = 2048
D_SPACE = 64
_POOLS = (1024, 1024, 1024, 1024, 1024, 1024, 2048, 2048)
_EMB_OFF = (0, 1024, 2048, 3072, 4096, 5120, 6144, 8192)
_TOTAL_EMB = 10240
_NPROJ = 8 * D_SPACE
_TM = 128


def _router_body(x_ref, wa_ref, wf_ref, wr_ref, b_ref, emb_ref, *refs):
    out_refs = refs[:8]
    normb_ref = refs[8]
    wb_ref = refs[9]
    i = pl.program_id(0)

    @pl.when(i == 0)
    def _():
        e = emb_ref[...]
        ss = jnp.sum(e * e, axis=1, keepdims=True)
        normb_ref[...] = (e / jnp.maximum(jnp.sqrt(ss), 1e-12)).astype(jnp.bfloat16)
        wb_ref[0:6 * D_SPACE, :] = wa_ref[...].astype(jnp.bfloat16)
        wb_ref[6 * D_SPACE:7 * D_SPACE, :] = wf_ref[...].astype(jnp.bfloat16)
        wb_ref[7 * D_SPACE:8 * D_SPACE, :] = wr_ref[...].astype(jnp.bfloat16)

    h = jax.lax.dot_general(
        x_ref[...].astype(jnp.bfloat16), wb_ref[...], (((1,), (1,)), ((), ())),
        preferred_element_type=jnp.float32) + b_ref[...]
    hb = h.astype(jnp.bfloat16)
    for p in range(8):
        hp = hb[:, p * D_SPACE:(p + 1) * D_SPACE]
        ep = normb_ref[_EMB_OFF[p]:_EMB_OFF[p] + _POOLS[p], :]
        out_refs[p][...] = jax.lax.dot_general(
            hp, ep, (((1,), (1,)), ((), ())),
            preferred_element_type=jnp.float32)


def kernel(x, W_all, b_all, W_fk, b_fk, W_rk, b_rk, neuron_emb):
    B, S, D = x.shape
    T = B * S
    xf = x.reshape(T, D)
    bc = jnp.concatenate([b_all, b_fk, b_rk])[None, :]

    grid = (T // _TM,)
    outs = pl.pallas_call(
        _router_body,
        grid=grid,
        in_specs=[
            pl.BlockSpec((_TM, D_MODEL), lambda i: (i, 0)),
            pl.BlockSpec((6 * D_SPACE, D_MODEL), lambda i: (0, 0)),
            pl.BlockSpec((D_SPACE, D_MODEL), lambda i: (0, 0)),
            pl.BlockSpec((D_SPACE, D_MODEL), lambda i: (0, 0)),
            pl.BlockSpec((1, _NPROJ), lambda i: (0, 0)),
            pl.BlockSpec((_TOTAL_EMB, D_SPACE), lambda i: (0, 0)),
        ],
        out_specs=[pl.BlockSpec((_TM, n), lambda i: (i, 0)) for n in _POOLS],
        out_shape=[jax.ShapeDtypeStruct((T, n), jnp.float32) for n in _POOLS],
        scratch_shapes=[pltpu.VMEM((_TOTAL_EMB, D_SPACE), jnp.bfloat16),
                        pltpu.VMEM((_NPROJ, D_MODEL), jnp.bfloat16)],
    )(xf, W_all, W_fk, W_rk, bc, neuron_emb)
    return tuple(o.reshape(B, S, n) for o, n in zip(outs, _POOLS))


# speedup vs baseline: 1.1393x; 1.1393x over previous
import jax
import jax.numpy as jnp
from jax.experimental import pallas as pl
from jax.experimental.pallas import tpu as pltpu

D_MODEL = 2048
D_SPACE = 64
_POOLS = (1024, 1024, 1024, 1024, 1024, 1024, 2048, 2048)
_EMB_OFF = (0, 1024, 2048, 3072, 4096, 5120, 6144, 8192)
_TOTAL_EMB = 10240
_NPROJ = 8 * D_SPACE
_TM = 256


def _router_body(x_ref, wa_ref, wf_ref, wr_ref, b_ref, emb_ref, *refs):
    out_refs = refs[:8]
    normb_ref = refs[8]
    wb_ref = refs[9]
    i = pl.program_id(0)

    @pl.when(i == 0)
    def _():
        e = emb_ref[...]
        ss = jnp.sum(e * e, axis=1, keepdims=True)
        normb_ref[...] = (e / jnp.maximum(jnp.sqrt(ss), 1e-12)).astype(jnp.bfloat16)
        wb_ref[0:6 * D_SPACE, :] = wa_ref[...].astype(jnp.bfloat16)
        wb_ref[6 * D_SPACE:7 * D_SPACE, :] = wf_ref[...].astype(jnp.bfloat16)
        wb_ref[7 * D_SPACE:8 * D_SPACE, :] = wr_ref[...].astype(jnp.bfloat16)

    h = jax.lax.dot_general(
        x_ref[...].astype(jnp.bfloat16), wb_ref[...], (((1,), (1,)), ((), ())),
        preferred_element_type=jnp.float32) + b_ref[...]
    hb = h.astype(jnp.bfloat16)
    for p in range(8):
        hp = hb[:, p * D_SPACE:(p + 1) * D_SPACE]
        ep = normb_ref[_EMB_OFF[p]:_EMB_OFF[p] + _POOLS[p], :]
        out_refs[p][...] = jax.lax.dot_general(
            hp, ep, (((1,), (1,)), ((), ())),
            preferred_element_type=jnp.float32)


def kernel(x, W_all, b_all, W_fk, b_fk, W_rk, b_rk, neuron_emb):
    B, S, D = x.shape
    T = B * S
    xf = x.reshape(T, D)
    bc = jnp.concatenate([b_all, b_fk, b_rk])[None, :]

    grid = (T // _TM,)
    outs = pl.pallas_call(
        _router_body,
        grid=grid,
        in_specs=[
            pl.BlockSpec((_TM, D_MODEL), lambda i: (i, 0)),
            pl.BlockSpec((6 * D_SPACE, D_MODEL), lambda i: (0, 0)),
            pl.BlockSpec((D_SPACE, D_MODEL), lambda i: (0, 0)),
            pl.BlockSpec((D_SPACE, D_MODEL), lambda i: (0, 0)),
            pl.BlockSpec((1, _NPROJ), lambda i: (0, 0)),
            pl.BlockSpec((_TOTAL_EMB, D_SPACE), lambda i: (0, 0)),
        ],
        out_specs=[pl.BlockSpec((_TM, n), lambda i: (i, 0)) for n in _POOLS],
        out_shape=[jax.ShapeDtypeStruct((T, n), jnp.float32) for n in _POOLS],
        scratch_shapes=[pltpu.VMEM((_TOTAL_EMB, D_SPACE), jnp.bfloat16),
                        pltpu.VMEM((_NPROJ, D_MODEL), jnp.bfloat16)],
    )(xf, W_all, W_fk, W_rk, bc, neuron_emb)
    return tuple(o.reshape(B, S, n) for o, n in zip(outs, _POOLS))
